# baseline (speedup 1.0000x reference)
import functools

import jax
import jax.numpy as jnp
from jax import lax
from jax.experimental import pallas as pl
from jax.experimental.pallas import tpu as pltpu
from jax.experimental.pallas import tpu_sc as plsc

N = 10000
NP = 10112
E = 320000
NW = 32
CH = 128
TPC = 80
EP = NW * TPC * CH
RS = NP // 16
NB = 3

_mesh = plsc.VectorSubcoreMesh(core_axis_name="c", subcore_axis_name="s")
_sc_params = pltpu.CompilerParams(use_tc_tiling_on_sc=False)


@functools.partial(
    pl.kernel,
    out_type=jax.ShapeDtypeStruct((2, NP, 16), jnp.float32),
    mesh=_mesh,
    compiler_params=pltpu.CompilerParams(use_tc_tiling_on_sc=True),
    scratch_types=[
        pltpu.VMEM((TPC, CH), jnp.int32),
        pltpu.VMEM((CH, 16), jnp.float32),
        pltpu.VMEM_SHARED((NP, 16), jnp.float32),
    ],
)
def _deg_sc(dst_hbm, zeros_hbm, ones_hbm, out_hbm, dstv, onesv, deg_sp):
    c = lax.axis_index("c")
    s = lax.axis_index("s")
    w = c * 16 + s
    pltpu.sync_copy(zeros_hbm.at[pl.ds(s * RS, RS)], deg_sp.at[pl.ds(s * RS, RS)])
    pltpu.sync_copy(ones_hbm, onesv)
    pltpu.sync_copy(dst_hbm.at[w], dstv)
    plsc.subcore_barrier()

    def body(j, carry):
        pltpu.sync_copy(onesv, deg_sp.at[dstv.at[j]], add=True)
        return carry

    lax.fori_loop(0, TPC, body, 0)
    plsc.subcore_barrier()
    pltpu.sync_copy(deg_sp.at[pl.ds(s * RS, RS)], out_hbm.at[c, pl.ds(s * RS, RS)])


@functools.partial(
    pl.kernel,
    out_type=jax.ShapeDtypeStruct((2, NP, 64), jnp.float32),
    mesh=_mesh,
    compiler_params=_sc_params,
    scratch_types=[
        pltpu.VMEM((TPC, CH), jnp.int32),
        pltpu.VMEM((TPC, CH), jnp.int32),
        pltpu.VMEM((NB, CH, 64), jnp.float32),
        pltpu.VMEM_SHARED((NP, 64), jnp.float32),
        pltpu.VMEM_SHARED((NP, 64), jnp.float32),
        pltpu.SemaphoreType.DMA,
        pltpu.SemaphoreType.DMA,
        pltpu.SemaphoreType.DMA,
    ],
)
def _agg_sc(hs_hbm, src_hbm, dst_hbm, zeros_hbm, out_hbm,
            srcv, dstv, rows, agg_sp, hs_sp, gsem0, gsem1, gsem2):
    gsems = (gsem0, gsem1, gsem2)
    c = lax.axis_index("c")
    s = lax.axis_index("s")
    w = c * 16 + s
    stripe = pl.ds(s * RS, RS)

    pltpu.sync_copy(zeros_hbm.at[stripe], agg_sp.at[stripe])
    pltpu.sync_copy(hs_hbm.at[stripe], hs_sp.at[stripe])
    pltpu.sync_copy(src_hbm.at[w], srcv)
    pltpu.sync_copy(dst_hbm.at[w], dstv)
    plsc.subcore_barrier()

    def gstart(j, b):
        pltpu.make_async_copy(hs_sp.at[srcv.at[j]], rows.at[b], gsems[b]).start()

    def gwait(j, b):
        pltpu.make_async_copy(hs_sp.at[srcv.at[j]], rows.at[b], gsems[b]).wait()

    def scat(j, b):
        pltpu.sync_copy(rows.at[b], agg_sp.at[dstv.at[j]], add=True)

    gstart(0, 0)
    gstart(1, 1)

    def body(i, carry):
        for b in range(NB):
            j = i * NB + b
            gwait(j, b)

            @pl.when(j + 2 < TPC)
            def _():
                gstart(j + 2, (b + 2) % NB)

            scat(j, b)
        return carry

    lax.fori_loop(0, TPC // NB, body, 0)
    for j in range((TPC // NB) * NB, TPC):
        gwait(j, j % NB)
        scat(j, j % NB)
    plsc.subcore_barrier()
    pltpu.sync_copy(agg_sp.at[stripe], out_hbm.at[c, stripe])


def _tc_mm1(x_ref, w_ref, t_ref):
    t_ref[...] = jnp.dot(x_ref[...], w_ref[...],
                         preferred_element_type=jnp.float32)


def _tc_first(t_ref, degp_ref, hs_ref, dinv_ref):
    deg = degp_ref[0, :, 0:1] + degp_ref[1, :, 0:1] + 1.0
    rows = lax.broadcasted_iota(jnp.int32, (NP, 1), 0)
    dinv = jnp.where(rows < N, lax.rsqrt(deg), 0.0)
    hs_ref[:N] = t_ref[...] * dinv[:N]
    hs_ref[N:] = jnp.zeros((NP - N, 64), jnp.float32)
    dinv_ref[...] = jnp.broadcast_to(dinv, (NP, 16))


def _tc_mid(p_ref, hs_ref, dinv_ref, b_ref, w_ref, out_ref):
    dinv = dinv_ref[:, 0:1]
    agg = p_ref[0] + p_ref[1] + hs_ref[...]
    h = jnp.maximum(agg * dinv + b_ref[...], 0.0)
    out_ref[...] = jnp.dot(h, w_ref[...], preferred_element_type=jnp.float32) * dinv


def _tc_pre4(p_ref, hs_ref, dinv_ref, b_ref, out_ref):
    dinv = dinv_ref[:, 0:1]
    agg = p_ref[0] + p_ref[1] + hs_ref[...]
    out_ref[...] = jnp.maximum(agg * dinv + b_ref[...], 0.0) * dinv


def _tc_last(p_ref, hs_ref, dinv_ref, w_ref, b_ref, out_ref):
    dinv = dinv_ref[:N, 0:1]
    agg = (p_ref[0, :N] + p_ref[1, :N] + hs_ref[:N]) * dinv
    out_ref[...] = jnp.dot(agg, w_ref[...], preferred_element_type=jnp.float32) + b_ref[...]


def _f32(*shape):
    return jax.ShapeDtypeStruct(shape, jnp.float32)


def kernel(x, edge_index, W1, b1, W2, b2, W3, b3, W4, b4):
    f32 = jnp.float32
    pad = EP - E
    srcb = jnp.concatenate(
        [edge_index[0], jnp.zeros((pad,), jnp.int32)]).reshape(NW, TPC, CH)
    dstb = jnp.concatenate(
        [edge_index[1], jnp.full((pad,), N, jnp.int32)]).reshape(NW, TPC, CH)
    zeros64 = jnp.zeros((NP, 64), f32)
    zeros16 = jnp.zeros((NP, 16), f32)
    ones16 = jnp.ones((CH, 16), f32)

    t1 = pl.pallas_call(_tc_mm1, out_shape=_f32(N, 64))(x, W1)
    degp = _deg_sc(dstb, zeros16, ones16)
    hs1, dinv16 = pl.pallas_call(
        _tc_first, out_shape=(_f32(NP, 64), _f32(NP, 16)))(t1, degp)
    p1 = _agg_sc(hs1, srcb, dstb, zeros64)
    hs2 = pl.pallas_call(_tc_mid, out_shape=_f32(NP, 64))(
        p1, hs1, dinv16, b1.reshape(1, 64), W2)
    p2 = _agg_sc(hs2, srcb, dstb, zeros64)
    hs3 = pl.pallas_call(_tc_mid, out_shape=_f32(NP, 64))(
        p2, hs2, dinv16, b2.reshape(1, 64), W3)
    p3 = _agg_sc(hs3, srcb, dstb, zeros64)
    hs4 = pl.pallas_call(_tc_pre4, out_shape=_f32(NP, 64))(
        p3, hs3, dinv16, b3.reshape(1, 64))
    p4 = _agg_sc(hs4, srcb, dstb, zeros64)
    out = pl.pallas_call(_tc_last, out_shape=_f32(N, 128))(
        p4, hs4, dinv16, W4, b4.reshape(1, 128))
    return out


# speedup vs baseline: 1.0102x; 1.0102x over previous
import functools

import jax
import jax.numpy as jnp
from jax import lax
from jax.experimental import pallas as pl
from jax.experimental.pallas import tpu as pltpu
from jax.experimental.pallas import tpu_sc as plsc

N = 10000
NP = 10112
E = 320000
NW = 32
CH = 128
TPC = 80
EP = NW * TPC * CH
RS = NP // 16
NB = 3

_mesh = plsc.VectorSubcoreMesh(core_axis_name="c", subcore_axis_name="s")
_sc_params = pltpu.CompilerParams(use_tc_tiling_on_sc=False)


@functools.partial(
    pl.kernel,
    out_type=jax.ShapeDtypeStruct((2, NP, 16), jnp.float32),
    mesh=_mesh,
    compiler_params=_sc_params,
    scratch_types=[
        pltpu.VMEM((TPC, CH), jnp.int32),
        pltpu.VMEM((CH, 16), jnp.float32),
        pltpu.VMEM_SHARED((NP, 16), jnp.float32),
    ],
)
def _deg_sc(dst_hbm, zeros_hbm, ones_hbm, out_hbm, dstv, onesv, deg_sp):
    c = lax.axis_index("c")
    s = lax.axis_index("s")
    w = c * 16 + s
    pltpu.sync_copy(zeros_hbm.at[pl.ds(s * RS, RS)], deg_sp.at[pl.ds(s * RS, RS)])
    pltpu.sync_copy(ones_hbm, onesv)
    pltpu.sync_copy(dst_hbm.at[w], dstv)
    plsc.subcore_barrier()

    def body(j, carry):
        pltpu.sync_copy(onesv, deg_sp.at[dstv.at[j]], add=True)
        return carry

    lax.fori_loop(0, TPC, body, 0)
    plsc.subcore_barrier()
    pltpu.sync_copy(deg_sp.at[pl.ds(s * RS, RS)], out_hbm.at[c, pl.ds(s * RS, RS)])


@functools.partial(
    pl.kernel,
    out_type=jax.ShapeDtypeStruct((2, NP, 64), jnp.float32),
    mesh=_mesh,
    compiler_params=_sc_params,
    scratch_types=[
        pltpu.VMEM((TPC, CH), jnp.int32),
        pltpu.VMEM((TPC, CH), jnp.int32),
        pltpu.VMEM((NB, CH, 64), jnp.float32),
        pltpu.VMEM_SHARED((NP, 64), jnp.float32),
        pltpu.VMEM_SHARED((NP, 64), jnp.float32),
        pltpu.SemaphoreType.DMA,
        pltpu.SemaphoreType.DMA,
        pltpu.SemaphoreType.DMA,
    ],
)
def _agg_sc(hs_hbm, src_hbm, dst_hbm, zeros_hbm, out_hbm,
            srcv, dstv, rows, agg_sp, hs_sp, gsem0, gsem1, gsem2):
    gsems = (gsem0, gsem1, gsem2)
    c = lax.axis_index("c")
    s = lax.axis_index("s")
    w = c * 16 + s
    stripe = pl.ds(s * RS, RS)

    pltpu.sync_copy(zeros_hbm.at[stripe], agg_sp.at[stripe])
    pltpu.sync_copy(hs_hbm.at[stripe], hs_sp.at[stripe])
    pltpu.sync_copy(src_hbm.at[w], srcv)
    pltpu.sync_copy(dst_hbm.at[w], dstv)
    plsc.subcore_barrier()

    def gstart(j, b):
        pltpu.make_async_copy(hs_sp.at[srcv.at[j]], rows.at[b], gsems[b]).start()

    def gwait(j, b):
        pltpu.make_async_copy(hs_sp.at[srcv.at[j]], rows.at[b], gsems[b]).wait()

    def scat(j, b):
        pltpu.sync_copy(rows.at[b], agg_sp.at[dstv.at[j]], add=True)

    gstart(0, 0)
    gstart(1, 1)

    def body(i, carry):
        for b in range(NB):
            j = i * NB + b
            gwait(j, b)

            @pl.when(j + 2 < TPC)
            def _():
                gstart(j + 2, (b + 2) % NB)

            scat(j, b)
        return carry

    lax.fori_loop(0, TPC // NB, body, 0)
    for j in range((TPC // NB) * NB, TPC):
        gwait(j, j % NB)
        scat(j, j % NB)
    plsc.subcore_barrier()
    pltpu.sync_copy(agg_sp.at[stripe], out_hbm.at[c, stripe])


def _tc_first(x_ref, w_ref, degp_ref, hs_ref, dinv_ref):
    deg = degp_ref[0, :, 0:1] + degp_ref[1, :, 0:1] + 1.0
    rows = lax.broadcasted_iota(jnp.int32, (NP, 1), 0)
    dinv = jnp.where(rows < N, lax.rsqrt(deg), 0.0)
    t = jnp.dot(x_ref[...], w_ref[...], preferred_element_type=jnp.float32)
    hs_ref[:N] = t * dinv[:N]
    hs_ref[N:] = jnp.zeros((NP - N, 64), jnp.float32)
    dinv_ref[...] = jnp.broadcast_to(dinv, (NP, 16))


def _tc_mid(p_ref, hs_ref, dinv_ref, b_ref, w_ref, out_ref):
    dinv = dinv_ref[:, 0:1]
    agg = p_ref[0] + p_ref[1] + hs_ref[...]
    h = jnp.maximum(agg * dinv + b_ref[...], 0.0)
    out_ref[...] = jnp.dot(h, w_ref[...], preferred_element_type=jnp.float32) * dinv


def _tc_pre4(p_ref, hs_ref, dinv_ref, b_ref, out_ref):
    dinv = dinv_ref[:, 0:1]
    agg = p_ref[0] + p_ref[1] + hs_ref[...]
    out_ref[...] = jnp.maximum(agg * dinv + b_ref[...], 0.0) * dinv


def _tc_last(p_ref, hs_ref, dinv_ref, w_ref, b_ref, out_ref):
    dinv = dinv_ref[:N, 0:1]
    agg = (p_ref[0, :N] + p_ref[1, :N] + hs_ref[:N]) * dinv
    out_ref[...] = jnp.dot(agg, w_ref[...], preferred_element_type=jnp.float32) + b_ref[...]


def _f32(*shape):
    return jax.ShapeDtypeStruct(shape, jnp.float32)


_GB = 8
_BM = NP // _GB


def _mid_call(p, hs, dinv, b, w):
    return pl.pallas_call(
        _tc_mid,
        grid=(_GB,),
        in_specs=[
            pl.BlockSpec((2, _BM, 64), lambda i: (0, i, 0)),
            pl.BlockSpec((_BM, 64), lambda i: (i, 0)),
            pl.BlockSpec((_BM, 16), lambda i: (i, 0)),
            pl.BlockSpec((1, 64), lambda i: (0, 0)),
            pl.BlockSpec((64, 64), lambda i: (0, 0)),
        ],
        out_specs=pl.BlockSpec((_BM, 64), lambda i: (i, 0)),
        out_shape=_f32(NP, 64),
    )(p, hs, dinv, b, w)


def _pre4_call(p, hs, dinv, b):
    return pl.pallas_call(
        _tc_pre4,
        grid=(_GB,),
        in_specs=[
            pl.BlockSpec((2, _BM, 64), lambda i: (0, i, 0)),
            pl.BlockSpec((_BM, 64), lambda i: (i, 0)),
            pl.BlockSpec((_BM, 16), lambda i: (i, 0)),
            pl.BlockSpec((1, 64), lambda i: (0, 0)),
        ],
        out_specs=pl.BlockSpec((_BM, 64), lambda i: (i, 0)),
        out_shape=_f32(NP, 64),
    )(p, hs, dinv, b)


def kernel(x, edge_index, W1, b1, W2, b2, W3, b3, W4, b4):
    f32 = jnp.float32
    pad = EP - E
    srcb = jnp.concatenate(
        [edge_index[0], jnp.zeros((pad,), jnp.int32)]).reshape(NW, TPC, CH)
    dstb = jnp.concatenate(
        [edge_index[1], jnp.full((pad,), N, jnp.int32)]).reshape(NW, TPC, CH)
    zeros64 = jnp.zeros((NP, 64), f32)
    zeros16 = jnp.zeros((NP, 16), f32)
    ones16 = jnp.ones((CH, 16), f32)

    degp = _deg_sc(dstb, zeros16, ones16)
    hs1, dinv16 = pl.pallas_call(
        _tc_first, out_shape=(_f32(NP, 64), _f32(NP, 16)))(x, W1, degp)
    p1 = _agg_sc(hs1, srcb, dstb, zeros64)
    hs2 = _mid_call(p1, hs1, dinv16, b1.reshape(1, 64), W2)
    p2 = _agg_sc(hs2, srcb, dstb, zeros64)
    hs3 = _mid_call(p2, hs2, dinv16, b2.reshape(1, 64), W3)
    p3 = _agg_sc(hs3, srcb, dstb, zeros64)
    hs4 = _pre4_call(p3, hs3, dinv16, b3.reshape(1, 64))
    p4 = _agg_sc(hs4, srcb, dstb, zeros64)
    out = pl.pallas_call(_tc_last, out_shape=_f32(N, 128))(
        p4, hs4, dinv16, W4, b4.reshape(1, 128))
    return out


# speedup vs baseline: 1.0172x; 1.0069x over previous
import functools

import jax
import jax.numpy as jnp
from jax import lax
from jax.experimental import pallas as pl
from jax.experimental.pallas import tpu as pltpu
from jax.experimental.pallas import tpu_sc as plsc

N = 10000
NP = 10112
E = 320000
NW = 32
CH = 128
TPC = 80
EP = NW * TPC * CH
RS = NP // 16
NB = 3

_mesh = plsc.VectorSubcoreMesh(core_axis_name="c", subcore_axis_name="s")
_sc_params = pltpu.CompilerParams(use_tc_tiling_on_sc=False)


@functools.partial(
    pl.kernel,
    out_type=jax.ShapeDtypeStruct((2, NP, 16), jnp.float32),
    mesh=_mesh,
    compiler_params=_sc_params,
    scratch_types=[
        pltpu.VMEM((TPC, CH), jnp.int32),
        pltpu.VMEM((CH, 16), jnp.float32),
        pltpu.VMEM_SHARED((NP, 16), jnp.float32),
    ],
)
def _deg_sc(dst_hbm, zeros_hbm, ones_hbm, out_hbm, dstv, onesv, deg_sp):
    c = lax.axis_index("c")
    s = lax.axis_index("s")
    w = c * 16 + s
    pltpu.sync_copy(zeros_hbm.at[pl.ds(s * RS, RS)], deg_sp.at[pl.ds(s * RS, RS)])
    pltpu.sync_copy(ones_hbm, onesv)
    pltpu.sync_copy(dst_hbm.at[w], dstv)
    plsc.subcore_barrier()

    def body(j, carry):
        pltpu.sync_copy(onesv, deg_sp.at[dstv.at[j]], add=True)
        return carry

    lax.fori_loop(0, TPC, body, 0)
    plsc.subcore_barrier()
    pltpu.sync_copy(deg_sp.at[pl.ds(s * RS, RS)], out_hbm.at[c, pl.ds(s * RS, RS)])


@functools.partial(
    pl.kernel,
    out_type=jax.ShapeDtypeStruct((2, NP, 64), jnp.float32),
    mesh=_mesh,
    compiler_params=_sc_params,
    scratch_types=[
        pltpu.VMEM((TPC, CH), jnp.int32),
        pltpu.VMEM((TPC, CH), jnp.int32),
        pltpu.VMEM((NB, CH, 64), jnp.float32),
        pltpu.VMEM_SHARED((NP, 64), jnp.float32),
        pltpu.VMEM_SHARED((NP, 64), jnp.float32),
        pltpu.SemaphoreType.DMA,
        pltpu.SemaphoreType.DMA,
        pltpu.SemaphoreType.DMA,
    ],
)
def _agg_sc(hs_hbm, src_hbm, dst_hbm, zeros_hbm, out_hbm,
            srcv, dstv, rows, agg_sp, hs_sp, gsem0, gsem1, gsem2):
    gsems = (gsem0, gsem1, gsem2)
    c = lax.axis_index("c")
    s = lax.axis_index("s")
    w = c * 16 + s
    stripe = pl.ds(s * RS, RS)

    @pl.when(c == 0)
    def _():
        pltpu.sync_copy(hs_hbm.at[stripe], agg_sp.at[stripe])

    @pl.when(c == 1)
    def _():
        pltpu.sync_copy(zeros_hbm.at[stripe], agg_sp.at[stripe])

    pltpu.sync_copy(hs_hbm.at[stripe], hs_sp.at[stripe])
    pltpu.sync_copy(src_hbm.at[w], srcv)
    pltpu.sync_copy(dst_hbm.at[w], dstv)
    plsc.subcore_barrier()

    def gstart(j, b):
        pltpu.make_async_copy(hs_sp.at[srcv.at[j]], rows.at[b], gsems[b]).start()

    def gwait(j, b):
        pltpu.make_async_copy(hs_sp.at[srcv.at[j]], rows.at[b], gsems[b]).wait()

    def scat(j, b):
        pltpu.sync_copy(rows.at[b], agg_sp.at[dstv.at[j]], add=True)

    gstart(0, 0)
    gstart(1, 1)

    def body(i, carry):
        for b in range(NB):
            j = i * NB + b
            gwait(j, b)

            @pl.when(j + 2 < TPC)
            def _():
                gstart(j + 2, (b + 2) % NB)

            scat(j, b)
        return carry

    lax.fori_loop(0, TPC // NB, body, 0)
    for j in range((TPC // NB) * NB, TPC):
        gwait(j, j % NB)
        scat(j, j % NB)
    plsc.subcore_barrier()
    pltpu.sync_copy(agg_sp.at[stripe], out_hbm.at[c, stripe])


def _tc_first(x_ref, w_ref, degp_ref, hs_ref, dinv_ref):
    deg = degp_ref[0, :, 0:1] + degp_ref[1, :, 0:1] + 1.0
    rows = lax.broadcasted_iota(jnp.int32, (NP, 1), 0)
    dinv = jnp.where(rows < N, lax.rsqrt(deg), 0.0)
    t = jnp.dot(x_ref[...], w_ref[...], preferred_element_type=jnp.float32)
    hs_ref[:N] = t * dinv[:N]
    hs_ref[N:] = jnp.zeros((NP - N, 64), jnp.float32)
    dinv_ref[...] = jnp.broadcast_to(dinv, (NP, 16))


def _tc_mid(p_ref, dinv_ref, b_ref, w_ref, out_ref):
    dinv = dinv_ref[:, 0:1]
    agg = p_ref[0] + p_ref[1]
    h = jnp.maximum(agg * dinv + b_ref[...], 0.0)
    out_ref[...] = jnp.dot(h, w_ref[...], preferred_element_type=jnp.float32) * dinv


def _tc_pre4(p_ref, dinv_ref, b_ref, out_ref):
    dinv = dinv_ref[:, 0:1]
    agg = p_ref[0] + p_ref[1]
    out_ref[...] = jnp.maximum(agg * dinv + b_ref[...], 0.0) * dinv


def _tc_last(p_ref, dinv_ref, w_ref, b_ref, out_ref):
    dinv = dinv_ref[:N, 0:1]
    agg = (p_ref[0, :N] + p_ref[1, :N]) * dinv
    out_ref[...] = jnp.dot(agg, w_ref[...], preferred_element_type=jnp.float32) + b_ref[...]


def _f32(*shape):
    return jax.ShapeDtypeStruct(shape, jnp.float32)


_GB = 8
_BM = NP // _GB


def _mid_call(p, dinv, b, w):
    return pl.pallas_call(
        _tc_mid,
        grid=(_GB,),
        in_specs=[
            pl.BlockSpec((2, _BM, 64), lambda i: (0, i, 0)),
            pl.BlockSpec((_BM, 16), lambda i: (i, 0)),
            pl.BlockSpec((1, 64), lambda i: (0, 0)),
            pl.BlockSpec((64, 64), lambda i: (0, 0)),
        ],
        out_specs=pl.BlockSpec((_BM, 64), lambda i: (i, 0)),
        out_shape=_f32(NP, 64),
    )(p, dinv, b, w)


def _pre4_call(p, dinv, b):
    return pl.pallas_call(
        _tc_pre4,
        grid=(_GB,),
        in_specs=[
            pl.BlockSpec((2, _BM, 64), lambda i: (0, i, 0)),
            pl.BlockSpec((_BM, 16), lambda i: (i, 0)),
            pl.BlockSpec((1, 64), lambda i: (0, 0)),
        ],
        out_specs=pl.BlockSpec((_BM, 64), lambda i: (i, 0)),
        out_shape=_f32(NP, 64),
    )(p, dinv, b)


def kernel(x, edge_index, W1, b1, W2, b2, W3, b3, W4, b4):
    f32 = jnp.float32
    pad = EP - E
    srcb = jnp.concatenate(
        [edge_index[0], jnp.zeros((pad,), jnp.int32)]).reshape(NW, TPC, CH)
    dstb = jnp.concatenate(
        [edge_index[1], jnp.full((pad,), N, jnp.int32)]).reshape(NW, TPC, CH)
    zeros64 = jnp.zeros((NP, 64), f32)
    zeros16 = jnp.zeros((NP, 16), f32)
    ones16 = jnp.ones((CH, 16), f32)

    degp = _deg_sc(dstb, zeros16, ones16)
    hs1, dinv16 = pl.pallas_call(
        _tc_first, out_shape=(_f32(NP, 64), _f32(NP, 16)))(x, W1, degp)
    p1 = _agg_sc(hs1, srcb, dstb, zeros64)
    hs2 = _mid_call(p1, dinv16, b1.reshape(1, 64), W2)
    p2 = _agg_sc(hs2, srcb, dstb, zeros64)
    hs3 = _mid_call(p2, dinv16, b2.reshape(1, 64), W3)
    p3 = _agg_sc(hs3, srcb, dstb, zeros64)
    hs4 = _pre4_call(p3, dinv16, b3.reshape(1, 64))
    p4 = _agg_sc(hs4, srcb, dstb, zeros64)
    out = pl.pallas_call(_tc_last, out_shape=_f32(N, 128))(
        p4, dinv16, W4, b4.reshape(1, 128))
    return out
